# Initial kernel scaffold; baseline (speedup 1.0000x reference)
import functools

import jax
import jax.numpy as jnp
from jax import lax
from jax.experimental import pallas as pl
from jax.experimental.pallas import tpu as pltpu
from jax.experimental.pallas import tpu_sc as plsc

LANES = 16
CHUNK = 128
N_SC = 2
N_TILES = 16
HALF = 128


def _fill_const_2d(ref, rows, cols, val):
    per_row = cols // LANES

    def body(k, _):
        r = k // per_row
        col0 = (k % per_row) * LANES
        ref[r, pl.ds(col0, LANES)] = jnp.full((LANES,), val, jnp.float32)
        return 0

    lax.fori_loop(0, rows * per_row, body, 0)


def _deg_body(n_nodes, n_pad, chunks_per_tile, rows_per_tile,
              dst_hbm, deg_out, idx_v, ones_v, zeros_v, acc_sh):
    c = lax.axis_index("c")
    s = lax.axis_index("s")
    t = c * N_TILES + s
    pltpu.sync_copy(dst_hbm.at[pl.ds(t * chunks_per_tile, chunks_per_tile)], idx_v)
    _fill_const_2d(ones_v, CHUNK, LANES, 1.0)
    _fill_const_2d(zeros_v, CHUNK, LANES, 0.0)
    zrows = n_pad // N_TILES
    for z in range(zrows // CHUNK):
        pltpu.sync_copy(zeros_v, acc_sh.at[pl.ds(s * zrows + z * CHUNK, CHUNK)])
    plsc.subcore_barrier()

    def body(j, _):
        pltpu.sync_copy(ones_v, acc_sh.at[idx_v.at[j]], add=True)
        return 0

    lax.fori_loop(0, chunks_per_tile, body, 0)
    plsc.subcore_barrier()
    pltpu.sync_copy(acc_sh.at[pl.ds(s * rows_per_tile, rows_per_tile)],
                    deg_out.at[c, pl.ds(s * rows_per_tile, rows_per_tile)])


def _agg_body(n_nodes, n_pad, chunks_per_tile, rows_per_tile,
              srcs_hbm, dsts_hbm, y_hbm, agg_out,
              sidx_v, didx_v, buf0, buf1, zeros_v, acc_sh, sem0, sem1):
    c = lax.axis_index("c")
    s = lax.axis_index("s")
    pltpu.sync_copy(srcs_hbm.at[c, pl.ds(s * chunks_per_tile, chunks_per_tile)], sidx_v)
    pltpu.sync_copy(dsts_hbm.at[pl.ds(s * chunks_per_tile, chunks_per_tile)], didx_v)
    pltpu.make_async_copy(y_hbm.at[sidx_v.at[0]], buf0, sem0).start()
    pltpu.make_async_copy(y_hbm.at[sidx_v.at[1]], buf1, sem1).start()
    _fill_const_2d(zeros_v, CHUNK, HALF, 0.0)
    zrows = n_pad // N_TILES
    for z in range(zrows // CHUNK):
        pltpu.sync_copy(zeros_v, acc_sh.at[pl.ds(s * zrows + z * CHUNK, CHUNK)])
    plsc.subcore_barrier()

    bufs = ((buf0, sem0), (buf1, sem1))
    n_outer = chunks_per_tile // 2

    def outer(g, _):
        for bi, (buf, sem) in enumerate(bufs):
            j = 2 * g + bi
            pltpu.make_async_copy(y_hbm.at[sidx_v.at[j]], buf, sem).wait()
            pltpu.sync_copy(buf, acc_sh.at[didx_v.at[j]], add=True)

            @pl.when(g < n_outer - 1)
            def _():
                pltpu.make_async_copy(y_hbm.at[sidx_v.at[j + 2]], buf, sem).start()
        return 0

    lax.fori_loop(0, n_outer, outer, 0)
    plsc.subcore_barrier()
    pltpu.sync_copy(acc_sh.at[pl.ds(s * rows_per_tile, rows_per_tile)],
                    agg_out.at[c, pl.ds(s * rows_per_tile, rows_per_tile)])


def _mm_body(x_ref, w_ref, deg_ref, y_ref):
    xw = jnp.dot(x_ref[...], w_ref[...], preferred_element_type=jnp.float32)
    dsum = deg_ref[0, :, 0:1] + deg_ref[1, :, 0:1] + 1.0
    dinv = lax.rsqrt(dsum)
    y = xw * dinv
    y_ref[0] = y[:, :HALF]
    y_ref[1] = y[:, HALF:]


def _out_body(agg_ref, y_ref, deg_ref, b_ref, o_ref):
    dsum = deg_ref[0, :, 0:1] + deg_ref[1, :, 0:1] + 1.0
    dinv = lax.rsqrt(dsum)
    lo = dinv * (agg_ref[0] + y_ref[0]) + b_ref[0, :HALF][None, :]
    hi = dinv * (agg_ref[1] + y_ref[1]) + b_ref[0, HALF:][None, :]
    o_ref[...] = jnp.concatenate([lo, hi], axis=1)


def kernel(x, edge_index, W, b):
    n, in_ch = x.shape
    out_ch = W.shape[1]
    e = edge_index.shape[1]

    pad_unit = N_TILES * CHUNK
    n_pad = ((n + pad_unit - 1) // pad_unit) * pad_unit
    if n_pad == n:
        n_pad = n + pad_unit
    rows_per_tile = n // N_TILES
    e_align = N_SC * N_TILES * CHUNK
    e_pad = ((e + e_align - 1) // e_align) * e_align
    if e_pad == e:
        e_pad = e + e_align
    n_fill = e_pad - e

    src = edge_index[0]
    dst = edge_index[1]
    fill = jnp.arange(n_fill, dtype=jnp.int32)
    src_p = jnp.concatenate([src, fill % n])
    dst_p = jnp.concatenate([dst, n + fill % (n_pad - n)])
    srcs2 = jnp.stack([src_p, src_p + n]).reshape(N_SC, e_pad // CHUNK, CHUNK)
    dsts = dst_p.reshape(e_pad // CHUNK, CHUNK)

    mesh = plsc.VectorSubcoreMesh(core_axis_name="c", subcore_axis_name="s")

    deg_call = pl.kernel(
        functools.partial(_deg_body, n, n_pad, e_pad // (N_SC * N_TILES * CHUNK),
                          rows_per_tile),
        out_type=jax.ShapeDtypeStruct((N_SC, n, LANES), jnp.float32),
        scratch_types=[
            pltpu.VMEM((e_pad // (N_SC * N_TILES * CHUNK), CHUNK), jnp.int32),
            pltpu.VMEM((CHUNK, LANES), jnp.float32),
            pltpu.VMEM((CHUNK, LANES), jnp.float32),
            pltpu.VMEM_SHARED((n_pad, LANES), jnp.float32),
        ],
        mesh=mesh,
    )
    deg2 = deg_call(dsts)

    blk = 400
    grid = n // blk
    y2 = pl.pallas_call(
        _mm_body,
        grid=(grid,),
        in_specs=[
            pl.BlockSpec((blk, in_ch), lambda i: (i, 0)),
            pl.BlockSpec((in_ch, out_ch), lambda i: (0, 0)),
            pl.BlockSpec((N_SC, blk, LANES), lambda i: (0, i, 0)),
        ],
        out_specs=pl.BlockSpec((N_SC, blk, HALF), lambda i: (0, i, 0)),
        out_shape=jax.ShapeDtypeStruct((N_SC, n, HALF), jnp.float32),
    )(x, W, deg2)

    agg_call = pl.kernel(
        functools.partial(_agg_body, n, n_pad, e_pad // (N_TILES * CHUNK),
                          rows_per_tile),
        out_type=jax.ShapeDtypeStruct((N_SC, n, HALF), jnp.float32),
        scratch_types=[
            pltpu.VMEM((e_pad // (N_TILES * CHUNK), CHUNK), jnp.int32),
            pltpu.VMEM((e_pad // (N_TILES * CHUNK), CHUNK), jnp.int32),
            pltpu.VMEM((CHUNK, HALF), jnp.float32),
            pltpu.VMEM((CHUNK, HALF), jnp.float32),
            pltpu.VMEM((CHUNK, HALF), jnp.float32),
            pltpu.VMEM_SHARED((n_pad, HALF), jnp.float32),
            pltpu.SemaphoreType.DMA,
            pltpu.SemaphoreType.DMA,
        ],
        mesh=mesh,
    )
    agg2 = agg_call(srcs2, dsts, y2.reshape(N_SC * n, HALF))

    out = pl.pallas_call(
        _out_body,
        grid=(grid,),
        in_specs=[
            pl.BlockSpec((N_SC, blk, HALF), lambda i: (0, i, 0)),
            pl.BlockSpec((N_SC, blk, HALF), lambda i: (0, i, 0)),
            pl.BlockSpec((N_SC, blk, LANES), lambda i: (0, i, 0)),
            pl.BlockSpec((1, out_ch), lambda i: (0, 0)),
        ],
        out_specs=pl.BlockSpec((blk, out_ch), lambda i: (i, 0)),
        out_shape=jax.ShapeDtypeStruct((n, out_ch), jnp.float32),
    )(agg2, y2, deg2, b.reshape(1, out_ch))
    return out


# speedup vs baseline: 18.4230x; 18.4230x over previous
import functools

import jax
import jax.numpy as jnp
from jax import lax
from jax.experimental import pallas as pl
from jax.experimental.pallas import tpu as pltpu
from jax.experimental.pallas import tpu_sc as plsc

LANES = 16
CHUNK = 128
IDX_PHASES = 2
N_SC = 2
N_TILES = 16
HALF = 128


def _fill_const_2d(ref, rows, cols, val):
    per_row = cols // LANES

    def body(k, _):
        r = k // per_row
        col0 = (k % per_row) * LANES
        ref[r, pl.ds(col0, LANES)] = jnp.full((LANES,), val, jnp.float32)
        return 0

    lax.fori_loop(0, rows * per_row, body, 0)


def _deg_body(n_nodes, n_pad, chunks_per_tile,
              dst_hbm, deg_out, idx_v, ones_v, acc_sh):
    c = lax.axis_index("c")
    s = lax.axis_index("s")
    t = c * N_TILES + s
    pltpu.sync_copy(dst_hbm.at[pl.ds(t * chunks_per_tile, chunks_per_tile)], idx_v)
    _fill_const_2d(ones_v, CHUNK, HALF, 0.0)
    zrows = n_pad // N_TILES
    for z0 in range(0, zrows, CHUNK):
        zlen = min(CHUNK, zrows - z0)
        pltpu.sync_copy(ones_v.at[pl.ds(0, zlen)],
                        acc_sh.at[pl.ds(s * zrows + z0, zlen)])
    _fill_const_2d(ones_v, CHUNK, HALF, 1.0)
    plsc.subcore_barrier()

    def body(j, _):
        pltpu.sync_copy(ones_v, acc_sh.at[idx_v.at[j]], add=True)
        return 0

    lax.fori_loop(0, chunks_per_tile, body, 0)
    plsc.subcore_barrier()
    pltpu.sync_copy(acc_sh.at[pl.ds(s * zrows, zrows)],
                    deg_out.at[c, pl.ds(s * zrows, zrows)])


def _agg_body(n_nodes, n_pad, chunks_per_tile,
              srcs_hbm, dsts_hbm, y_hbm, agg_out,
              sidx_v, didx_v, buf0, buf1, acc_sh, sem0, sem1):
    c = lax.axis_index("c")
    s = lax.axis_index("s")
    _fill_const_2d(buf0, CHUNK, HALF, 0.0)
    zrows = n_pad // N_TILES
    for z0 in range(0, zrows, CHUNK):
        zlen = min(CHUNK, zrows - z0)
        pltpu.sync_copy(buf0.at[pl.ds(0, zlen)],
                        acc_sh.at[pl.ds(s * zrows + z0, zlen)])
    plsc.subcore_barrier()

    phase_chunks = chunks_per_tile // IDX_PHASES
    bufs = ((buf0, sem0), (buf1, sem1))
    n_outer = phase_chunks // 2
    for phase in range(IDX_PHASES):
        base = s * chunks_per_tile + phase * phase_chunks
        pltpu.sync_copy(srcs_hbm.at[c, pl.ds(base, phase_chunks)], sidx_v)
        pltpu.sync_copy(dsts_hbm.at[pl.ds(base, phase_chunks)], didx_v)
        pltpu.make_async_copy(y_hbm.at[sidx_v.at[0]], buf0, sem0).start()
        pltpu.make_async_copy(y_hbm.at[sidx_v.at[1]], buf1, sem1).start()

        def outer(g, _):
            for bi, (buf, sem) in enumerate(bufs):
                j = 2 * g + bi
                pltpu.make_async_copy(y_hbm.at[sidx_v.at[j]], buf, sem).wait()
                pltpu.sync_copy(buf, acc_sh.at[didx_v.at[j]], add=True)

                @pl.when(g < n_outer - 1)
                def _():
                    pltpu.make_async_copy(y_hbm.at[sidx_v.at[j + 2]], buf, sem).start()
            return 0

        lax.fori_loop(0, n_outer, outer, 0)
    plsc.subcore_barrier()
    pltpu.sync_copy(acc_sh.at[pl.ds(s * zrows, zrows)],
                    agg_out.at[c, pl.ds(s * zrows, zrows)])


def _mm_body(x_ref, w_ref, deg_ref, y_ref):
    xw = jnp.dot(x_ref[...], w_ref[...], preferred_element_type=jnp.float32)
    dsum = deg_ref[0, :, 0:1] + deg_ref[1, :, 0:1] + 1.0
    dinv = lax.rsqrt(dsum)
    y = xw * dinv
    y_ref[0] = y[:, :HALF]
    y_ref[1] = y[:, HALF:]


def _out_body(agg_ref, y_ref, deg_ref, b_ref, o_ref):
    dsum = deg_ref[0, :, 0:1] + deg_ref[1, :, 0:1] + 1.0
    dinv = lax.rsqrt(dsum)
    lo = dinv * (agg_ref[0] + y_ref[0]) + b_ref[0, :HALF][None, :]
    hi = dinv * (agg_ref[1] + y_ref[1]) + b_ref[0, HALF:][None, :]
    o_ref[...] = jnp.concatenate([lo, hi], axis=1)


def kernel(x, edge_index, W, b):
    n, in_ch = x.shape
    out_ch = W.shape[1]
    e = edge_index.shape[1]

    pad_unit = N_TILES * 8
    n_pad = ((n + pad_unit - 1) // pad_unit) * pad_unit
    if n_pad == n:
        n_pad = n + pad_unit
    e_align = N_SC * N_TILES * CHUNK
    e_pad = ((e + e_align - 1) // e_align) * e_align
    if e_pad == e:
        e_pad = e + e_align
    n_fill = e_pad - e

    src = edge_index[0]
    dst = edge_index[1]
    fill = jnp.arange(n_fill, dtype=jnp.int32)
    src_p = jnp.concatenate([src, fill % n])
    dst_p = jnp.concatenate([dst, n + fill % (n_pad - n)])
    srcs2 = jnp.stack([src_p, src_p + n]).reshape(N_SC, e_pad // CHUNK, CHUNK)
    dsts = dst_p.reshape(e_pad // CHUNK, CHUNK)

    mesh = plsc.VectorSubcoreMesh(core_axis_name="c", subcore_axis_name="s")

    deg_call = pl.kernel(
        functools.partial(_deg_body, n, n_pad, e_pad // (N_SC * N_TILES * CHUNK)),
        out_type=jax.ShapeDtypeStruct((N_SC, n_pad, HALF), jnp.float32),
        scratch_types=[
            pltpu.VMEM((e_pad // (N_SC * N_TILES * CHUNK), CHUNK), jnp.int32),
            pltpu.VMEM((CHUNK, HALF), jnp.float32),
            pltpu.VMEM_SHARED((n_pad, HALF), jnp.float32),
        ],
        mesh=mesh,
    )
    deg2 = deg_call(dsts)

    blk = 400
    grid = n // blk
    y2 = pl.pallas_call(
        _mm_body,
        grid=(grid,),
        in_specs=[
            pl.BlockSpec((blk, in_ch), lambda i: (i, 0)),
            pl.BlockSpec((in_ch, out_ch), lambda i: (0, 0)),
            pl.BlockSpec((N_SC, blk, HALF), lambda i: (0, i, 0)),
        ],
        out_specs=pl.BlockSpec((N_SC, blk, HALF), lambda i: (0, i, 0)),
        out_shape=jax.ShapeDtypeStruct((N_SC, n, HALF), jnp.float32),
    )(x, W, deg2)

    chunks_per_tile = e_pad // (N_TILES * CHUNK)
    agg_call = pl.kernel(
        functools.partial(_agg_body, n, n_pad, chunks_per_tile),
        out_type=jax.ShapeDtypeStruct((N_SC, n_pad, HALF), jnp.float32),
        scratch_types=[
            pltpu.VMEM((chunks_per_tile // IDX_PHASES, CHUNK), jnp.int32),
            pltpu.VMEM((chunks_per_tile // IDX_PHASES, CHUNK), jnp.int32),
            pltpu.VMEM((CHUNK, HALF), jnp.float32),
            pltpu.VMEM((CHUNK, HALF), jnp.float32),
            pltpu.VMEM_SHARED((n_pad, HALF), jnp.float32),
            pltpu.SemaphoreType.DMA,
            pltpu.SemaphoreType.DMA,
        ],
        mesh=mesh,
    )
    agg2 = agg_call(srcs2, dsts, y2.reshape(N_SC * n, HALF))

    out = pl.pallas_call(
        _out_body,
        grid=(grid,),
        in_specs=[
            pl.BlockSpec((N_SC, blk, HALF), lambda i: (0, i, 0)),
            pl.BlockSpec((N_SC, blk, HALF), lambda i: (0, i, 0)),
            pl.BlockSpec((N_SC, blk, HALF), lambda i: (0, i, 0)),
            pl.BlockSpec((1, out_ch), lambda i: (0, 0)),
        ],
        out_specs=pl.BlockSpec((blk, out_ch), lambda i: (i, 0)),
        out_shape=jax.ShapeDtypeStruct((n, out_ch), jnp.float32),
    )(agg2, y2, deg2, b.reshape(1, out_ch))
    return out
